# Initial kernel scaffold; baseline (speedup 1.0000x reference)
import jax
import jax.numpy as jnp
from jax.experimental import pallas as pl


def kernel(q, k, v, PB, SE, phase_base, phase_gain, Wy, Wz):
    raise NotImplementedError("write your pallas kernel here")


# speedup vs baseline: 19.9356x; 19.9356x over previous
import numpy as np
import jax
import jax.numpy as jnp
from jax.experimental import pallas as pl

_SPARSE = [48, 64, 96, 128, 192, 256, 384, 512, 768, 1024, 1536]
_OFFS = tuple(list(range(33)) + _SPARSE)
_NOFF = len(_OFFS)
_NDENSE = 33
_NSPARSE = 11
_PAD = 1536


def _attn_body(qt_ref, kt_ref, vt_ref, pb_ref, se_ref, pbase_ref, pgain_ref,
               wy_ref, wz_ref, offs_ref, out_ref):
    qt = qt_ref[0]
    kt = kt_ref[0]
    vt = vt_ref[0]
    n = qt.shape[1]
    sc = 1.0 / np.sqrt(64.0)

    qse = jnp.dot(se_ref[...], qt, preferred_element_type=jnp.float32) * sc
    y_pre = jnp.dot(wy_ref[0].T, qt, preferred_element_type=jnp.float32)
    z_pre = jnp.dot(wz_ref[0].T, kt, preferred_element_type=jnp.float32)

    rows = []
    for d in _OFFS:
        ks = kt[:, _PAD - d:_PAD - d + n]
        rows.append(jnp.sum(qt * ks, axis=0, keepdims=True))
    s = jnp.concatenate(rows, axis=0) * sc
    s = s + qse + pb_ref[0]

    pos = jax.lax.broadcasted_iota(jnp.int32, (_NOFF, n), 1)
    valid = pos >= offs_ref[...]
    s = jnp.where(valid, s, -1e30)

    m = jnp.max(s, axis=0, keepdims=True)
    e = jnp.exp(s - m)
    alpha = e / jnp.sum(e, axis=0, keepdims=True)
    alpha = jnp.where(valid, alpha, 0.0)

    acc = jnp.zeros((64, n), dtype=jnp.float32)
    for i, d in enumerate(_OFFS):
        a = alpha[i:i + 1, :]
        vs = vt[:, _PAD - d:_PAD - d + n]
        if i < _NDENSE:
            acc = acc + a * vs
        else:
            j = i - _NDENSE
            z0 = z_pre[0:1, _PAD - d:_PAD - d + n]
            z1 = z_pre[1:2, _PAD - d:_PAD - d + n]
            th0 = pbase_ref[0, j:j + 1, 0:1] + pgain_ref[0, j:j + 1, 0:1] * y_pre[0:1, :] * z0
            th1 = pbase_ref[0, j:j + 1, 1:2] + pgain_ref[0, j:j + 1, 1:2] * y_pre[1:2, :] * z1
            c0, s0 = jnp.cos(th0), jnp.sin(th0)
            c1, s1 = jnp.cos(th1), jnp.sin(th1)
            v0, v1 = vs[0:1, :], vs[1:2, :]
            v2, v3 = vs[2:3, :], vs[3:4, :]
            vrot = jnp.concatenate([
                c0 * v0 - s0 * v1,
                s0 * v0 + c0 * v1,
                c1 * v2 - s1 * v3,
                s1 * v2 + c1 * v3,
                vs[4:, :],
            ], axis=0)
            acc = acc + a * vrot
    out_ref[0] = acc


def kernel(q, k, v, PB, SE, phase_base, phase_gain, Wy, Wz):
    B, H, N, HD = q.shape
    qt = jnp.transpose(q[0], (0, 2, 1))
    kp = jnp.pad(k[0], ((0, 0), (_PAD, 0), (0, 0)))
    vp = jnp.pad(v[0], ((0, 0), (_PAD, 0), (0, 0)))
    kt = jnp.transpose(kp, (0, 2, 1))
    vt = jnp.transpose(vp, (0, 2, 1))
    pb3 = jnp.transpose(PB).reshape(H, _NOFF, 1)
    pbase = jnp.transpose(phase_base, (1, 0, 2))
    pgain = jnp.transpose(phase_gain, (1, 0, 2))

    out_t = pl.pallas_call(
        _attn_body,
        grid=(H,),
        in_specs=[
            pl.BlockSpec((1, HD, N), lambda h: (h, 0, 0)),
            pl.BlockSpec((1, HD, N + _PAD), lambda h: (h, 0, 0)),
            pl.BlockSpec((1, HD, N + _PAD), lambda h: (h, 0, 0)),
            pl.BlockSpec((1, _NOFF, 1), lambda h: (h, 0, 0)),
            pl.BlockSpec((_NOFF, HD), lambda h: (0, 0)),
            pl.BlockSpec((1, _NSPARSE, 2), lambda h: (h, 0, 0)),
            pl.BlockSpec((1, _NSPARSE, 2), lambda h: (h, 0, 0)),
            pl.BlockSpec((1, HD, 2), lambda h: (h, 0, 0)),
            pl.BlockSpec((1, HD, 2), lambda h: (h, 0, 0)),
            pl.BlockSpec((_NOFF, 1), lambda h: (0, 0)),
        ],
        out_specs=pl.BlockSpec((1, HD, N), lambda h: (h, 0, 0)),
        out_shape=jax.ShapeDtypeStruct((H, HD, N), jnp.float32),
    )
    offs = jnp.asarray(np.array(_OFFS, np.int32).reshape(_NOFF, 1))
    out_t = out_t(qt, kt, vt, pb3, SE, pbase, pgain, Wy, Wz, offs)

    return jnp.transpose(out_t, (0, 2, 1))[None]


# speedup vs baseline: 21.4603x; 1.0765x over previous
import numpy as np
import jax
import jax.numpy as jnp
from jax.experimental import pallas as pl

_SPARSE = [48, 64, 96, 128, 192, 256, 384, 512, 768, 1024, 1536]
_OFFS = tuple(list(range(33)) + _SPARSE)
_NOFF = len(_OFFS)
_NDENSE = 33
_NSPARSE = 11
_PAD = 1536
_CH = 512


def _attn_body(qt_ref, kt_ref, vt_ref, pb_ref, se_ref, pbase_ref, pgain_ref,
               wy_ref, wz_ref, offs_ref, out_ref):
    n = qt_ref.shape[2]
    sc = 1.0 / np.sqrt(64.0)
    qt = qt_ref[0] * sc
    kt = kt_ref[0]

    bias = jnp.dot(se_ref[...], qt, preferred_element_type=jnp.float32) + pb_ref[0]
    y_pre = jnp.dot(wy_ref[0].T, qt_ref[0], preferred_element_type=jnp.float32)
    z_pre = jnp.dot(wz_ref[0].T, kt, preferred_element_type=jnp.float32)

    for c in range(n // _CH):
        n0 = c * _CH
        qc = qt[:, n0:n0 + _CH]
        rows = []
        live = []
        for i, d in enumerate(_OFFS):
            if n0 + _CH <= d:
                continue
            ks = kt_ref[0, :, _PAD + n0 - d:_PAD + n0 - d + _CH]
            rows.append(jnp.sum(qc * ks, axis=0, keepdims=True))
            live.append(i)
        i0 = live[0]
        nlive = len(live)
        s = jnp.concatenate(rows, axis=0) + bias[i0:i0 + nlive, n0:n0 + _CH]

        pos = jax.lax.broadcasted_iota(jnp.int32, (nlive, _CH), 1) + n0
        valid = pos >= offs_ref[i0:i0 + nlive]
        s = jnp.where(valid, s, -1e30)

        m = jnp.max(s, axis=0, keepdims=True)
        e = jnp.exp(s - m)
        alpha = e * (1.0 / jnp.sum(e, axis=0, keepdims=True))

        acc = jnp.zeros((64, _CH), dtype=jnp.float32)
        for r, i in enumerate(live):
            d = _OFFS[i]
            a = alpha[r:r + 1, :]
            vs = vt_ref[0, :, _PAD + n0 - d:_PAD + n0 - d + _CH]
            if i < _NDENSE:
                acc = acc + a * vs
            else:
                j = i - _NDENSE
                z0 = z_pre[0:1, _PAD + n0 - d:_PAD + n0 - d + _CH]
                z1 = z_pre[1:2, _PAD + n0 - d:_PAD + n0 - d + _CH]
                th0 = pbase_ref[0, j:j + 1, 0:1] + pgain_ref[0, j:j + 1, 0:1] * y_pre[0:1, n0:n0 + _CH] * z0
                th1 = pbase_ref[0, j:j + 1, 1:2] + pgain_ref[0, j:j + 1, 1:2] * y_pre[1:2, n0:n0 + _CH] * z1
                c0, s0 = jnp.cos(th0), jnp.sin(th0)
                c1, s1 = jnp.cos(th1), jnp.sin(th1)
                v0, v1 = vs[0:1, :], vs[1:2, :]
                v2, v3 = vs[2:3, :], vs[3:4, :]
                vrot = jnp.concatenate([
                    c0 * v0 - s0 * v1,
                    s0 * v0 + c0 * v1,
                    c1 * v2 - s1 * v3,
                    s1 * v2 + c1 * v3,
                    vs[4:, :],
                ], axis=0)
                acc = acc + a * vrot
        out_ref[0, :, n0:n0 + _CH] = acc


def kernel(q, k, v, PB, SE, phase_base, phase_gain, Wy, Wz):
    B, H, N, HD = q.shape
    qt = jnp.transpose(q[0], (0, 2, 1))
    kp = jnp.pad(k[0], ((0, 0), (_PAD, 0), (0, 0)))
    vp = jnp.pad(v[0], ((0, 0), (_PAD, 0), (0, 0)))
    kt = jnp.transpose(kp, (0, 2, 1))
    vt = jnp.transpose(vp, (0, 2, 1))
    pb3 = jnp.transpose(PB).reshape(H, _NOFF, 1)
    pbase = jnp.transpose(phase_base, (1, 0, 2))
    pgain = jnp.transpose(phase_gain, (1, 0, 2))

    grid_call = pl.pallas_call(
        _attn_body,
        grid=(H,),
        in_specs=[
            pl.BlockSpec((1, HD, N), lambda h: (h, 0, 0)),
            pl.BlockSpec((1, HD, N + _PAD), lambda h: (h, 0, 0)),
            pl.BlockSpec((1, HD, N + _PAD), lambda h: (h, 0, 0)),
            pl.BlockSpec((1, _NOFF, 1), lambda h: (h, 0, 0)),
            pl.BlockSpec((_NOFF, HD), lambda h: (0, 0)),
            pl.BlockSpec((1, _NSPARSE, 2), lambda h: (h, 0, 0)),
            pl.BlockSpec((1, _NSPARSE, 2), lambda h: (h, 0, 0)),
            pl.BlockSpec((1, HD, 2), lambda h: (h, 0, 0)),
            pl.BlockSpec((1, HD, 2), lambda h: (h, 0, 0)),
            pl.BlockSpec((_NOFF, 1), lambda h: (0, 0)),
        ],
        out_specs=pl.BlockSpec((1, HD, N), lambda h: (h, 0, 0)),
        out_shape=jax.ShapeDtypeStruct((H, HD, N), jnp.float32),
    )
    offs = jnp.asarray(np.array(_OFFS, np.int32).reshape(_NOFF, 1))
    out_t = grid_call(qt, kt, vt, pb3, SE, pbase, pgain, Wy, Wz, offs)

    return jnp.transpose(out_t, (0, 2, 1))[None]
